# Initial kernel scaffold; baseline (speedup 1.0000x reference)
import jax
import jax.numpy as jnp
from jax.experimental import pallas as pl


def kernel(x, Wg, bg, Wn, bn, W1, b1, W2, b2):
    raise NotImplementedError("write your pallas kernel here")


# speedup vs baseline: 1.1792x; 1.1792x over previous
import functools

import jax
import jax.numpy as jnp
from jax import lax
from jax.experimental import pallas as pl
from jax.experimental.pallas import tpu as pltpu
from jax.experimental.pallas import tpu_sc as plsc

_N_EMBED = 1024
_NUM_EXPERTS = 8
_TOP_K = 2
_BM = 256


def _router(x, Wg, bg, Wn, bn):
    logits = x @ Wg + bg
    safe_noise = jax.nn.softplus(x @ Wn + bn)
    eps = jax.random.normal(jax.random.key(42), logits.shape, dtype=logits.dtype)
    noisy_logits = logits + eps * safe_noise
    top_v, top_i = lax.top_k(noisy_logits, _TOP_K)
    probs = jax.nn.softmax(top_v, axis=-1)
    return top_i, probs


def _dispatch(top_i, probs, n_tokens):
    E, BM, K = _NUM_EXPERTS, _BM, _TOP_K
    A = n_tokens * K
    P = A + E * BM

    flat_e = top_i.reshape(-1).astype(jnp.int32)
    sort_idx = jnp.argsort(flat_e).astype(jnp.int32)
    sorted_e = flat_e[sort_idx]
    counts = jnp.bincount(flat_e, length=E).astype(jnp.int32)
    padded = ((counts + BM - 1) // BM) * BM
    pstart = jnp.concatenate([jnp.zeros((1,), jnp.int32),
                              jnp.cumsum(padded)[:-1].astype(jnp.int32)])
    start = jnp.concatenate([jnp.zeros((1,), jnp.int32),
                             jnp.cumsum(counts)[:-1].astype(jnp.int32)])
    dst_sorted = pstart[sorted_e] + (jnp.arange(A, dtype=jnp.int32)
                                     - start[sorted_e])
    src_assign = jnp.zeros((P,), jnp.int32).at[dst_sorted].set(sort_idx)
    valid = jnp.zeros((P,), jnp.bool_).at[dst_sorted].set(True)
    src_token = src_assign // K
    ws = jnp.where(valid, probs.reshape(-1)[src_assign], 0.0)
    n_m = P // BM
    gids = (jnp.searchsorted(pstart,
                             jnp.arange(n_m, dtype=jnp.int32) * BM,
                             side="right").astype(jnp.int32) - 1)
    dst_orig = jnp.zeros((A,), jnp.int32).at[sort_idx].set(dst_sorted)
    pos = dst_orig.reshape(n_tokens, K)
    return src_token, ws, gids, pos, P


def _sc_gather(table, idx):
    P = idx.shape[0]
    D = table.shape[1]
    info = plsc.get_sparse_core_info()
    NW = info.num_cores * info.num_subcores
    rows_per_w = P // NW
    CH = 64
    n_ch = rows_per_w // CH
    mesh = plsc.VectorSubcoreMesh(core_axis_name="c", subcore_axis_name="s")

    @functools.partial(
        pl.kernel,
        mesh=mesh,
        out_type=jax.ShapeDtypeStruct((P, D), jnp.float32),
        scratch_types=[
            pltpu.VMEM((CH,), jnp.int32),
            pltpu.VMEM((CH, D), jnp.float32),
            pltpu.SemaphoreType.DMA,
        ],
    )
    def k(table_hbm, idx_hbm, out_hbm, idx_v, rows_v, sem):
        wid = lax.axis_index("s") * info.num_cores + lax.axis_index("c")
        base = wid * rows_per_w

        def body(i, carry):
            off = base + i * CH
            pltpu.sync_copy(idx_hbm.at[pl.ds(off, CH)], idx_v)
            pltpu.async_copy(table_hbm.at[idx_v], rows_v, sem).wait()
            pltpu.sync_copy(rows_v, out_hbm.at[pl.ds(off, CH)])
            return carry

        lax.fori_loop(0, n_ch, body, 0)

    return k(table, idx)


def _sc_combine(ys, pos0, pos1):
    N = pos0.shape[0]
    D = ys.shape[1]
    info = plsc.get_sparse_core_info()
    NW = info.num_cores * info.num_subcores
    rows_per_w = N // NW
    CH = 32
    n_ch = rows_per_w // CH
    L = 16
    mesh = plsc.VectorSubcoreMesh(core_axis_name="c", subcore_axis_name="s")

    @functools.partial(
        pl.kernel,
        mesh=mesh,
        out_type=jax.ShapeDtypeStruct((N, D), jnp.float32),
        scratch_types=[
            pltpu.VMEM((CH,), jnp.int32),
            pltpu.VMEM((CH, D), jnp.float32),
            pltpu.VMEM((CH, D), jnp.float32),
            pltpu.SemaphoreType.DMA,
        ],
    )
    def k(ys_hbm, p0_hbm, p1_hbm, out_hbm, idx_v, a_v, b_v, sem):
        wid = lax.axis_index("s") * info.num_cores + lax.axis_index("c")
        base = wid * rows_per_w

        def body(i, carry):
            off = base + i * CH
            pltpu.sync_copy(p0_hbm.at[pl.ds(off, CH)], idx_v)
            pltpu.async_copy(ys_hbm.at[idx_v], a_v, sem).wait()
            pltpu.sync_copy(p1_hbm.at[pl.ds(off, CH)], idx_v)
            pltpu.async_copy(ys_hbm.at[idx_v], b_v, sem).wait()

            def add_row(r, c2):
                def add_vec(j, c3):
                    a_v[r, pl.ds(j * L, L)] = (a_v[r, pl.ds(j * L, L)]
                                               + b_v[r, pl.ds(j * L, L)])
                    return c3
                lax.fori_loop(0, D // L, add_vec, 0)
                return c2

            lax.fori_loop(0, CH, add_row, 0)
            pltpu.sync_copy(a_v, out_hbm.at[pl.ds(off, CH)])
            return carry

        lax.fori_loop(0, n_ch, body, 0)

    return k(ys, pos0, pos1)


def _ffn1(gids, xsb, W1b, b1):
    P, D = xsb.shape
    E, _, H = W1b.shape
    n_m = P // _BM

    def kern(g_ref, xs_ref, w1_ref, b1_ref, o_ref):
        acc = jnp.dot(xs_ref[...], w1_ref[0],
                      preferred_element_type=jnp.float32)
        acc = jnp.maximum(acc + b1_ref[0], 0.0)
        o_ref[...] = acc.astype(jnp.bfloat16)

    grid_spec = pltpu.PrefetchScalarGridSpec(
        num_scalar_prefetch=1,
        grid=(n_m,),
        in_specs=[
            pl.BlockSpec((_BM, D), lambda i, g: (i, 0)),
            pl.BlockSpec((1, D, H), lambda i, g: (g[i], 0, 0)),
            pl.BlockSpec((1, 1, H), lambda i, g: (g[i], 0, 0)),
        ],
        out_specs=pl.BlockSpec((_BM, H), lambda i, g: (i, 0)),
    )
    return pl.pallas_call(
        kern,
        grid_spec=grid_spec,
        out_shape=jax.ShapeDtypeStruct((P, H), jnp.bfloat16),
        compiler_params=pltpu.CompilerParams(
            dimension_semantics=("arbitrary",)),
    )(gids, xsb, W1b, b1)


def _ffn2(gids, h, W2b, b2, ws):
    P, H = h.shape
    E, _, D = W2b.shape
    n_m = P // _BM

    def kern(g_ref, h_ref, w2_ref, b2_ref, ws_ref, o_ref):
        acc = jnp.dot(h_ref[...], w2_ref[0],
                      preferred_element_type=jnp.float32)
        o_ref[...] = (acc + b2_ref[0]) * ws_ref[...]

    grid_spec = pltpu.PrefetchScalarGridSpec(
        num_scalar_prefetch=1,
        grid=(n_m,),
        in_specs=[
            pl.BlockSpec((_BM, H), lambda i, g: (i, 0)),
            pl.BlockSpec((1, H, D), lambda i, g: (g[i], 0, 0)),
            pl.BlockSpec((1, 1, D), lambda i, g: (g[i], 0, 0)),
            pl.BlockSpec((_BM, 1), lambda i, g: (i, 0)),
        ],
        out_specs=pl.BlockSpec((_BM, D), lambda i, g: (i, 0)),
    )
    return pl.pallas_call(
        kern,
        grid_spec=grid_spec,
        out_shape=jax.ShapeDtypeStruct((P, D), jnp.float32),
        compiler_params=pltpu.CompilerParams(
            dimension_semantics=("arbitrary",)),
    )(gids, h, W2b, b2, ws)


def kernel(x, Wg, bg, Wn, bn, W1, b1, W2, b2):
    B, S, D = x.shape
    N = B * S
    top_i, probs = _router(x, Wg, bg, Wn, bn)
    src_token, ws, gids, pos, P = _dispatch(top_i, probs, N)

    x2d = x.reshape(N, D)
    xs = _sc_gather(x2d, src_token)
    E, _, H = W1.shape
    h = _ffn1(gids, xs.astype(jnp.bfloat16), W1.astype(jnp.bfloat16),
              b1.reshape(E, 1, H))
    ys = _ffn2(gids, h, W2.astype(jnp.bfloat16), b2.reshape(E, 1, D),
               ws.reshape(P, 1))
    out2d = _sc_combine(ys, pos[:, 0], pos[:, 1])
    return out2d.reshape(B, S, D)
